# Initial kernel scaffold; baseline (speedup 1.0000x reference)
import jax
import jax.numpy as jnp
from jax.experimental import pallas as pl


def kernel(x, adj, gcn1_w, gcn1_b, gcn2_w, gcn2_b, fc1_w, fc1_b, fc2_w, fc2_b):
    raise NotImplementedError("write your pallas kernel here")


# speedup vs baseline: 2.7227x; 2.7227x over previous
import functools

import jax
import jax.numpy as jnp
from jax.experimental import pallas as pl
from jax.experimental.pallas import tpu as pltpu

_NG = 16
_NN = 32
_DIN = 128
_DH = 64
_TB = 32
_GRP = 4
_ACH = 256


def _round_up(v, m):
    return (v + m - 1) // m * m


def _fused_body(x_ref, w1b_ref, a8_ref, b1t_ref, wbig_ref, beff_ref,
                wf2_ref, bf2_ref, o_ref):
    rows = _TB * _NN

    s_parts = []
    for q in range(_NG // _GRP):
        xcat = jnp.concatenate(
            [x_ref[:, _GRP * q + j].reshape(rows, _DIN) for j in range(_GRP)],
            axis=1)
        s_parts.append(jnp.dot(xcat, w1b_ref[q],
                               preferred_element_type=jnp.float32))
    s1 = jnp.concatenate(s_parts, axis=1)

    a8 = a8_ref[...]
    h = jnp.concatenate(
        [jnp.dot(a8, s1[c * _ACH:(c + 1) * _ACH],
                 preferred_element_type=jnp.float32)
         for c in range(rows // _ACH)], axis=0)
    h = jnp.maximum(h + b1t_ref[...], 0.0)
    h3 = h.reshape(_TB, _NN, _NG * _DH)

    acc = jnp.dot(h3[:, 0, :], wbig_ref[0],
                  preferred_element_type=jnp.float32)
    for m in range(1, _NN):
        acc = acc + jnp.dot(h3[:, m, :], wbig_ref[m],
                            preferred_element_type=jnp.float32)
    p = jnp.maximum(acc + beff_ref[...], 0.0)

    o_ref[...] = (jnp.dot(p, wf2_ref[...],
                          preferred_element_type=jnp.float32) + bf2_ref[...])


@functools.partial(jax.jit, static_argnames=())
def kernel(x, adj, gcn1_w, gcn1_b, gcn2_w, gcn2_b, fc1_w, fc1_b, fc2_w, fc2_b):
    B = x.shape[0]
    dense = fc1_w.shape[0]
    nclass = fc2_w.shape[0]

    xr = x.reshape(B, _NG, _NN, _DIN)
    b_pad = _round_up(max(B, _TB), _TB)
    if b_pad != B:
        xr = jnp.pad(xr, ((0, b_pad - B), (0, 0), (0, 0), (0, 0)))

    w1b = jnp.zeros((_NG // _GRP, _GRP * _DIN, _GRP * _DH), jnp.float32)
    for g in range(_NG):
        q, j = divmod(g, _GRP)
        w1b = w1b.at[q, j * _DIN:(j + 1) * _DIN, j * _DH:(j + 1) * _DH].set(
            gcn1_w[g])

    a8 = jnp.kron(jnp.eye(_ACH // _NN, dtype=adj.dtype), adj)
    b1t = jnp.tile(gcn1_b, _NG).reshape(1, _NG * _DH)

    wf1r = fc1_w.reshape(dense, _NG, _NN, 16)
    c1 = jnp.einsum('dgno,nm->dgmo', wf1r, adj)
    wbig = jnp.einsum('dgmo,gko->mgkd', c1, gcn2_w)
    wbig = wbig.reshape(_NN, _NG * _DH, dense)
    beff = (fc1_b + jnp.einsum('dgno,o->d', wf1r, gcn2_b)).reshape(1, dense)

    wf2 = jnp.pad(fc2_w.T, ((0, 0), (0, 128 - nclass)))
    bf2 = jnp.pad(fc2_b.reshape(1, nclass), ((0, 0), (0, 128 - nclass)))

    out = pl.pallas_call(
        _fused_body,
        out_shape=jax.ShapeDtypeStruct((b_pad, 128), jnp.float32),
        grid_spec=pltpu.PrefetchScalarGridSpec(
            num_scalar_prefetch=0,
            grid=(b_pad // _TB,),
            in_specs=[
                pl.BlockSpec((_TB, _NG, _NN, _DIN), lambda i: (i, 0, 0, 0)),
                pl.BlockSpec(w1b.shape, lambda i: (0, 0, 0)),
                pl.BlockSpec(a8.shape, lambda i: (0, 0)),
                pl.BlockSpec(b1t.shape, lambda i: (0, 0)),
                pl.BlockSpec(wbig.shape, lambda i: (0, 0, 0)),
                pl.BlockSpec(beff.shape, lambda i: (0, 0)),
                pl.BlockSpec(wf2.shape, lambda i: (0, 0)),
                pl.BlockSpec(bf2.shape, lambda i: (0, 0)),
            ],
            out_specs=pl.BlockSpec((_TB, 128), lambda i: (i, 0)),
        ),
        compiler_params=pltpu.CompilerParams(
            dimension_semantics=("parallel",),
            vmem_limit_bytes=60 * 1024 * 1024),
    )(xr, w1b, a8, b1t, wbig, beff, wf2, bf2)

    return out[:B, :nclass]
